# baseline (speedup 1.0000x reference)
import functools
import math

import jax
import jax.numpy as jnp
from jax import lax
from jax.experimental import pallas as pl
from jax.experimental.pallas import tpu as pltpu
from jax.experimental.pallas import tpu_sc as plsc

_SPARSITY = 0.05
_L = 16
_INT_MIN = -2147483648


def _splat(v, dtype=jnp.int32):
    return jnp.full((_L,), v, dtype=dtype)


def _to_key(i):
    return jnp.where(i < 0, i ^ jnp.int32(0x7FFFFFFF), i)


def _zero256(ref):
    z = jnp.zeros((_L,), jnp.int32)
    for c in range(16):
        ref[pl.ds(c * _L, _L)] = z


def _locate(hist, sufr, rank):
    ids0 = lax.iota(jnp.int32, _L)
    rank_s = _splat(rank)

    def p1(i, st):
        b_best, carry = st
        c = 15 - i
        h = hist[pl.ds(c * _L, _L)]
        suf = jnp.flip(jnp.cumsum(jnp.flip(h))) + _splat(carry)
        sufr[pl.ds(c * _L, _L)] = suf
        ids = ids0 + c * _L
        b_best = jnp.maximum(
            b_best, jnp.max(jnp.where(suf >= rank_s, ids, _splat(-1))))
        return b_best, carry + jnp.sum(h)

    b_best, _ = lax.fori_loop(0, 16, p1, (jnp.int32(-1), jnp.int32(0)))
    c = b_best >> 4
    h = hist[pl.ds(c * _L, _L)]
    suf = sufr[pl.ds(c * _L, _L)]
    sel = (ids0 + c * _L) == _splat(b_best)
    above = jnp.max(jnp.where(sel, suf - h, _splat(-1)))
    h_at = jnp.max(jnp.where(sel, h, _splat(-1)))
    return b_best, above, h_at


def _sc_body(n_chunks, k_active, x_hbm, out_hbm, xv, kv, cand, hist, sufr):
    wid = lax.axis_index("s") * 2 + lax.axis_index("c")
    base_row = wid * 2
    pltpu.sync_copy(x_hbm.at[pl.ds(base_row, 2)], xv)

    ones = jnp.ones((_L,), jnp.int32)
    ids0 = lax.iota(jnp.int32, _L)

    for r in range(2):
        _zero256(hist)

        def pass_a(c, _):
            xi = lax.bitcast_convert_type(xv[r, pl.ds(c * _L, _L)], jnp.int32)
            key = _to_key(xi)
            kv[r, pl.ds(c * _L, _L)] = key
            d1 = (key >> 24) + 128
            plsc.addupdate_scatter(hist, [d1], ones)
            return 0

        lax.fori_loop(0, n_chunks, pass_a, 0)
        b1, above1, h1 = _locate(hist, sufr, jnp.int32(k_active))
        rank2 = jnp.int32(k_active) - above1

        b1s = _splat(b1)

        def pass_b(c, off):
            key = kv[r, pl.ds(c * _L, _L)]
            m = ((key >> 24) + 128) == b1s
            plsc.store_compressed(cand.at[pl.ds(off, _L)], key, mask=m)
            return off + jnp.sum(m.astype(jnp.int32))

        n2 = lax.fori_loop(0, n_chunks, pass_b, jnp.int32(0))
        nc2 = (n2 + _L - 1) >> 4
        n2s = _splat(n2)

        def hist_level(shift, prev):
            _zero256(hist)

            def body(c, _):
                base = c * _L
                key = cand[pl.ds(base, _L)]
                m = (ids0 + base) < n2s
                for sh, b in prev:
                    m = jnp.logical_and(m, ((key >> sh) & 255) == _splat(b))
                d = (key >> shift) & 255
                plsc.addupdate_scatter(hist, [d], ones, mask=m)
                return 0

            lax.fori_loop(0, nc2, body, 0)

        hist_level(16, [])
        b2, above2, h2 = _locate(hist, sufr, rank2)
        rank3 = rank2 - above2

        hist_level(8, [(16, b2)])
        b3, above3, h3 = _locate(hist, sufr, rank3)
        rank4 = rank3 - above3

        hist_level(0, [(16, b2), (8, b3)])
        b4, above4, h4 = _locate(hist, sufr, rank4)

        v_key = ((b1 - 128) << 24) | (b2 << 16) | (b3 << 8) | b4
        c_ge = above1 + above2 + above3 + above4 + h4

        vks = _splat(v_key)

        def pass_c(c, mx):
            key = kv[r, pl.ds(c * _L, _L)]
            return jnp.maximum(mx, jnp.where(key < vks, key, _splat(_INT_MIN)))

        mx = lax.fori_loop(0, n_chunks, pass_c, _splat(_INT_MIN))
        v2_key = jnp.where(c_ge >= jnp.int32(k_active + 1), v_key, jnp.max(mx))

        f1 = lax.bitcast_convert_type(_to_key(vks), jnp.float32)
        f2 = lax.bitcast_convert_type(_to_key(_splat(v2_key)), jnp.float32)
        thr = (f1 + f2) * jnp.float32(0.5)
        tkey = _to_key(lax.bitcast_convert_type(thr, jnp.int32))

        def pass_d(c, _):
            key = kv[r, pl.ds(c * _L, _L)]
            xv[r, pl.ds(c * _L, _L)] = jnp.where(
                key > tkey, jnp.float32(1.0), jnp.float32(0.0))
            return 0

        lax.fori_loop(0, n_chunks, pass_d, 0)

    pltpu.sync_copy(xv, out_hbm.at[pl.ds(base_row, 2)])


def kernel(x):
    batch, emb = x.shape
    k_active = math.ceil(_SPARSITY * emb)
    rows_per_worker = batch // 32
    n_chunks = emb // _L
    mesh = plsc.VectorSubcoreMesh(core_axis_name="c", subcore_axis_name="s")
    body = functools.partial(_sc_body, n_chunks, k_active)
    f = pl.kernel(
        body,
        mesh=mesh,
        compiler_params=pltpu.CompilerParams(needs_layout_passes=False),
        out_type=jax.ShapeDtypeStruct((batch, emb), jnp.float32),
        scratch_types=[
            pltpu.VMEM((rows_per_worker, emb), jnp.float32),
            pltpu.VMEM((rows_per_worker, emb), jnp.int32),
            pltpu.VMEM((emb,), jnp.int32),
            pltpu.VMEM((256,), jnp.int32),
            pltpu.VMEM((256,), jnp.int32),
        ],
    )
    return f(x)


# speedup vs baseline: 1.0297x; 1.0297x over previous
import functools
import math

import jax
import jax.numpy as jnp
from jax import lax
from jax.experimental import pallas as pl
from jax.experimental.pallas import tpu as pltpu
from jax.experimental.pallas import tpu_sc as plsc

_SPARSITY = 0.05
_L = 16
_INT_MIN = -2147483648


def _splat(v, dtype=jnp.int32):
    return jnp.full((_L,), v, dtype=dtype)


def _to_key(i):
    return jnp.where(i < 0, i ^ jnp.int32(0x7FFFFFFF), i)


def _zero256(ref):
    z = jnp.zeros((_L,), jnp.int32)
    for c in range(16):
        ref[pl.ds(c * _L, _L)] = z


def _locate(hist, sufr, rank):
    ids0 = lax.iota(jnp.int32, _L)
    rank_s = _splat(rank)

    def p1(i, st):
        b_best, carry = st
        c = 15 - i
        h = hist[pl.ds(c * _L, _L)]
        suf = jnp.flip(jnp.cumsum(jnp.flip(h))) + _splat(carry)
        sufr[pl.ds(c * _L, _L)] = suf
        ids = ids0 + c * _L
        b_best = jnp.maximum(
            b_best, jnp.max(jnp.where(suf >= rank_s, ids, _splat(-1))))
        return b_best, carry + jnp.sum(h)

    b_best, _ = lax.fori_loop(0, 16, p1, (jnp.int32(-1), jnp.int32(0)))
    c = b_best >> 4
    h = hist[pl.ds(c * _L, _L)]
    suf = sufr[pl.ds(c * _L, _L)]
    sel = (ids0 + c * _L) == _splat(b_best)
    above = jnp.max(jnp.where(sel, suf - h, _splat(-1)))
    h_at = jnp.max(jnp.where(sel, h, _splat(-1)))
    return b_best, above, h_at


def _sc_body(n_chunks, k_active, x_hbm, out_hbm, xv, kv, cand, hist, sufr):
    wid = lax.axis_index("s") * 2 + lax.axis_index("c")
    base_row = wid * 2
    pltpu.sync_copy(x_hbm.at[pl.ds(base_row, 2)], xv)

    ones = jnp.ones((_L,), jnp.int32)
    ids0 = lax.iota(jnp.int32, _L)

    U = 8

    for r in range(2):
        _zero256(hist)

        def pass_a(c, _):
            for u in range(U):
                s = c * (_L * U) + u * _L
                xi = lax.bitcast_convert_type(xv[r, pl.ds(s, _L)], jnp.int32)
                key = _to_key(xi)
                kv[r, pl.ds(s, _L)] = key
                d1 = (key >> 24) + 128
                plsc.addupdate_scatter(hist, [d1], ones)
            return 0

        lax.fori_loop(0, n_chunks // U, pass_a, 0)
        b1, above1, h1 = _locate(hist, sufr, jnp.int32(k_active))
        rank2 = jnp.int32(k_active) - above1

        b1s = _splat(b1)

        def pass_b(c, off):
            for u in range(U):
                s = c * (_L * U) + u * _L
                key = kv[r, pl.ds(s, _L)]
                m = ((key >> 24) + 128) == b1s
                plsc.store_compressed(cand.at[pl.ds(off, _L)], key, mask=m)
                off = off + jnp.sum(m.astype(jnp.int32))
            return off

        n2 = lax.fori_loop(0, n_chunks // U, pass_b, jnp.int32(0))
        nc2 = (n2 + _L - 1) >> 4
        n2s = _splat(n2)

        def hist_level(shift, prev):
            _zero256(hist)

            def body(c, _):
                base = c * _L
                key = cand[pl.ds(base, _L)]
                m = (ids0 + base) < n2s
                for sh, b in prev:
                    m = jnp.logical_and(m, ((key >> sh) & 255) == _splat(b))
                d = (key >> shift) & 255
                plsc.addupdate_scatter(hist, [d], ones, mask=m)
                return 0

            lax.fori_loop(0, nc2, body, 0)

        hist_level(16, [])
        b2, above2, h2 = _locate(hist, sufr, rank2)
        rank3 = rank2 - above2

        hist_level(8, [(16, b2)])
        b3, above3, h3 = _locate(hist, sufr, rank3)
        rank4 = rank3 - above3

        hist_level(0, [(16, b2), (8, b3)])
        b4, above4, h4 = _locate(hist, sufr, rank4)

        v_key = ((b1 - 128) << 24) | (b2 << 16) | (b3 << 8) | b4
        c_ge = above1 + above2 + above3 + above4 + h4

        vks = _splat(v_key)

        def scan_cand():
            def body(c, mx):
                base = c * _L
                key = cand[pl.ds(base, _L)]
                m = jnp.logical_and((ids0 + base) < n2s, key < vks)
                return jnp.maximum(mx, jnp.where(m, key, _splat(_INT_MIN)))

            return lax.fori_loop(0, nc2, body, _splat(_INT_MIN))

        def scan_full():
            def body(c, mx):
                for u in range(U):
                    s = c * (_L * U) + u * _L
                    key = kv[r, pl.ds(s, _L)]
                    mx = jnp.maximum(mx, jnp.where(key < vks, key, _splat(_INT_MIN)))
                return mx

            return lax.fori_loop(0, n_chunks // U, body, _splat(_INT_MIN))

        mx = lax.cond(rank2 < n2, scan_cand, scan_full)
        v2_key = jnp.where(c_ge >= jnp.int32(k_active + 1), v_key, jnp.max(mx))

        f1 = lax.bitcast_convert_type(_to_key(vks), jnp.float32)
        f2 = lax.bitcast_convert_type(_to_key(_splat(v2_key)), jnp.float32)
        thr = (f1 + f2) * jnp.float32(0.5)
        tkey = _to_key(lax.bitcast_convert_type(thr, jnp.int32))

        def pass_d(c, _):
            for u in range(U):
                s = c * (_L * U) + u * _L
                key = kv[r, pl.ds(s, _L)]
                xv[r, pl.ds(s, _L)] = jnp.where(
                    key > tkey, jnp.float32(1.0), jnp.float32(0.0))
            return 0

        lax.fori_loop(0, n_chunks // U, pass_d, 0)

    pltpu.sync_copy(xv, out_hbm.at[pl.ds(base_row, 2)])


def kernel(x):
    batch, emb = x.shape
    k_active = math.ceil(_SPARSITY * emb)
    rows_per_worker = batch // 32
    n_chunks = emb // _L
    mesh = plsc.VectorSubcoreMesh(core_axis_name="c", subcore_axis_name="s")
    body = functools.partial(_sc_body, n_chunks, k_active)
    f = pl.kernel(
        body,
        mesh=mesh,
        compiler_params=pltpu.CompilerParams(needs_layout_passes=False),
        out_type=jax.ShapeDtypeStruct((batch, emb), jnp.float32),
        scratch_types=[
            pltpu.VMEM((rows_per_worker, emb), jnp.float32),
            pltpu.VMEM((rows_per_worker, emb), jnp.int32),
            pltpu.VMEM((emb,), jnp.int32),
            pltpu.VMEM((256,), jnp.int32),
            pltpu.VMEM((256,), jnp.int32),
        ],
    )
    return f(x)


# speedup vs baseline: 1.7641x; 1.7133x over previous
import functools
import math

import jax
import jax.numpy as jnp
from jax import lax
from jax.experimental import pallas as pl
from jax.experimental.pallas import tpu as pltpu
from jax.experimental.pallas import tpu_sc as plsc

_SPARSITY = 0.05
_L = 16
_INT_MIN = -2147483648


def _splat(v, dtype=jnp.int32):
    return jnp.full((_L,), v, dtype=dtype)


def _to_key(i):
    return jnp.where(i < 0, i ^ jnp.int32(0x7FFFFFFF), i)


def _zero256(ref):
    z = jnp.zeros((_L,), jnp.int32)
    for c in range(16):
        ref[pl.ds(c * _L, _L)] = z


def _locate(hist, sufr, rank):
    ids0 = lax.iota(jnp.int32, _L)
    rank_s = _splat(rank)

    def p1(i, st):
        bb, carry = st
        c = 15 - i
        h = hist[pl.ds(c * _L, _L)]
        suf = jnp.flip(jnp.cumsum(jnp.flip(h))) + _splat(carry)
        sufr[pl.ds(c * _L, _L)] = suf
        ids = ids0 + c * _L
        bb = jnp.maximum(bb, jnp.where(suf >= rank_s, ids, _splat(-1)))
        return bb, carry + jnp.sum(h)

    bb, _ = lax.fori_loop(0, 16, p1, (_splat(-1), jnp.int32(0)))
    b_best = jnp.max(bb)
    c = b_best >> 4
    h = hist[pl.ds(c * _L, _L)]
    suf = sufr[pl.ds(c * _L, _L)]
    sel = (ids0 + c * _L) == _splat(b_best)
    above = jnp.max(jnp.where(sel, suf - h, _splat(-1)))
    h_at = jnp.max(jnp.where(sel, h, _splat(-1)))
    return b_best, above, h_at


def _sc_body(n_chunks, k_active, x_hbm, out_hbm, xv, kv, hist, sufr):
    wid = lax.axis_index("s") * 2 + lax.axis_index("c")
    base_row = wid * 2
    pltpu.sync_copy(x_hbm.at[pl.ds(base_row, 2)], xv)

    ones = jnp.ones((_L,), jnp.int32)

    for r in range(2):
        _zero256(hist)

        @plsc.parallel_loop(0, n_chunks, unroll=8)
        def _(c):
            s = c * _L
            xi = lax.bitcast_convert_type(xv[r, pl.ds(s, _L)], jnp.int32)
            key = _to_key(xi)
            kv[r, pl.ds(s, _L)] = key
            plsc.addupdate_scatter(hist, [(key >> 24) + 128], ones)

        b1, above1, _h1 = _locate(hist, sufr, jnp.int32(k_active))
        rank2 = jnp.int32(k_active) - above1

        p1v = b1 - 128

        def hist_level(prefix, pshift, dshift):
            _zero256(hist)
            ps = _splat(prefix)

            @plsc.parallel_loop(0, n_chunks, unroll=8)
            def _(c):
                key = kv[r, pl.ds(c * _L, _L)]
                m = (key >> pshift) == ps
                plsc.addupdate_scatter(
                    hist, [(key >> dshift) & 255], ones, mask=m)

        hist_level(p1v, 24, 16)
        b2, above2, _h2 = _locate(hist, sufr, rank2)
        rank3 = rank2 - above2
        p2v = (p1v << 8) | b2

        hist_level(p2v, 16, 8)
        b3, above3, _h3 = _locate(hist, sufr, rank3)
        rank4 = rank3 - above3
        p3v = (p2v << 8) | b3

        hist_level(p3v, 8, 0)
        b4, above4, h4 = _locate(hist, sufr, rank4)

        v_key = (p3v << 8) | b4
        c_ge = above1 + above2 + above3 + above4 + h4

        vks = _splat(v_key)

        @plsc.parallel_loop(0, n_chunks, unroll=8, carry=_splat(_INT_MIN))
        def mx(c, acc):
            key = kv[r, pl.ds(c * _L, _L)]
            return jnp.maximum(acc, jnp.where(key < vks, key, _splat(_INT_MIN)))

        v2_key = jnp.where(c_ge >= jnp.int32(k_active + 1), v_key, jnp.max(mx))

        f1 = lax.bitcast_convert_type(_to_key(vks), jnp.float32)
        f2 = lax.bitcast_convert_type(_to_key(_splat(v2_key)), jnp.float32)
        thr = (f1 + f2) * jnp.float32(0.5)
        tkey = _to_key(lax.bitcast_convert_type(thr, jnp.int32))

        @plsc.parallel_loop(0, n_chunks, unroll=8)
        def _(c):
            s = c * _L
            key = kv[r, pl.ds(s, _L)]
            xv[r, pl.ds(s, _L)] = jnp.where(
                key > tkey, jnp.float32(1.0), jnp.float32(0.0))

    pltpu.sync_copy(xv, out_hbm.at[pl.ds(base_row, 2)])


def kernel(x):
    batch, emb = x.shape
    k_active = math.ceil(_SPARSITY * emb)
    rows_per_worker = batch // 32
    n_chunks = emb // _L
    mesh = plsc.VectorSubcoreMesh(core_axis_name="c", subcore_axis_name="s")
    body = functools.partial(_sc_body, n_chunks, k_active)
    f = pl.kernel(
        body,
        mesh=mesh,
        compiler_params=pltpu.CompilerParams(needs_layout_passes=False),
        out_type=jax.ShapeDtypeStruct((batch, emb), jnp.float32),
        scratch_types=[
            pltpu.VMEM((rows_per_worker, emb), jnp.float32),
            pltpu.VMEM((rows_per_worker, emb), jnp.int32),
            pltpu.VMEM((256,), jnp.int32),
            pltpu.VMEM((256,), jnp.int32),
        ],
    )
    return f(x)


# speedup vs baseline: 1.7998x; 1.0202x over previous
import functools
import math

import jax
import jax.numpy as jnp
from jax import lax
from jax.experimental import pallas as pl
from jax.experimental.pallas import tpu as pltpu
from jax.experimental.pallas import tpu_sc as plsc

_SPARSITY = 0.05
_L = 16
_INT_MIN = -2147483648


def _splat(v, dtype=jnp.int32):
    return jnp.full((_L,), v, dtype=dtype)


def _to_key(i):
    return jnp.where(i < 0, i ^ jnp.int32(0x7FFFFFFF), i)


def _zero256(ref):
    z = jnp.zeros((_L,), jnp.int32)
    for c in range(16):
        ref[pl.ds(c * _L, _L)] = z


def _locate(hist, sufr, tots, rank):
    ids0 = lax.iota(jnp.int32, _L)
    lane0 = ids0 == _splat(0)

    @plsc.parallel_loop(0, 16, unroll=4)
    def _(c):
        h = hist[pl.ds(c * _L, _L)]
        suf = jnp.flip(jnp.cumsum(jnp.flip(h)))
        sufr[pl.ds(c * _L, _L)] = suf
        plsc.store_scatter(tots, [_splat(c)], suf, mask=lane0)

    tv = tots[...]
    ts = jnp.flip(jnp.cumsum(jnp.flip(tv)))
    above_ch = ts - tv
    rank_s = _splat(rank)
    c_best = jnp.max(jnp.where(ts >= rank_s, ids0, _splat(-1)))
    above_c = jnp.max(
        jnp.where(ids0 == _splat(c_best), above_ch, _splat(_INT_MIN)))
    suf_w = sufr[pl.ds(c_best * _L, _L)] + _splat(above_c)
    h = hist[pl.ds(c_best * _L, _L)]
    ids = ids0 + c_best * _L
    b_best = jnp.max(jnp.where(suf_w >= rank_s, ids, _splat(-1)))
    sel = ids == _splat(b_best)
    above = jnp.max(jnp.where(sel, suf_w - h, _splat(-1)))
    h_at = jnp.max(jnp.where(sel, h, _splat(-1)))
    return b_best, above, h_at


def _sc_body(n_chunks, k_active, x_hbm, out_hbm, xv, kv, hist, sufr, tots):
    wid = lax.axis_index("s") * 2 + lax.axis_index("c")
    base_row = wid * 2
    pltpu.sync_copy(x_hbm.at[pl.ds(base_row, 2)], xv)

    ones = jnp.ones((_L,), jnp.int32)

    for r in range(2):
        _zero256(hist)

        @plsc.parallel_loop(0, n_chunks, unroll=8)
        def _(c):
            s = c * _L
            xi = lax.bitcast_convert_type(xv[r, pl.ds(s, _L)], jnp.int32)
            key = _to_key(xi)
            kv[r, pl.ds(s, _L)] = key
            plsc.addupdate_scatter(hist, [(key >> 24) + 128], ones)

        b1, above1, _h1 = _locate(hist, sufr, tots, jnp.int32(k_active))
        rank2 = jnp.int32(k_active) - above1

        p1v = b1 - 128

        def hist_level(prefix, pshift, dshift):
            _zero256(hist)
            ps = _splat(prefix)

            @plsc.parallel_loop(0, n_chunks, unroll=8)
            def _(c):
                key = kv[r, pl.ds(c * _L, _L)]
                m = (key >> pshift) == ps
                plsc.addupdate_scatter(
                    hist, [(key >> dshift) & 255], ones, mask=m)

        hist_level(p1v, 24, 16)
        b2, above2, _h2 = _locate(hist, sufr, tots, rank2)
        rank3 = rank2 - above2
        p2v = (p1v << 8) | b2

        hist_level(p2v, 16, 8)
        b3, above3, _h3 = _locate(hist, sufr, tots, rank3)
        rank4 = rank3 - above3
        p3v = (p2v << 8) | b3

        hist_level(p3v, 8, 0)
        b4, above4, h4 = _locate(hist, sufr, tots, rank4)

        v_key = (p3v << 8) | b4
        c_ge = above1 + above2 + above3 + above4 + h4

        vks = _splat(v_key)

        @plsc.parallel_loop(0, n_chunks, unroll=8, carry=_splat(_INT_MIN))
        def mx(c, acc):
            key = kv[r, pl.ds(c * _L, _L)]
            return jnp.maximum(acc, jnp.where(key < vks, key, _splat(_INT_MIN)))

        v2_key = jnp.where(c_ge >= jnp.int32(k_active + 1), v_key, jnp.max(mx))

        f1 = lax.bitcast_convert_type(_to_key(vks), jnp.float32)
        f2 = lax.bitcast_convert_type(_to_key(_splat(v2_key)), jnp.float32)
        thr = (f1 + f2) * jnp.float32(0.5)
        tkey = _to_key(lax.bitcast_convert_type(thr, jnp.int32))

        @plsc.parallel_loop(0, n_chunks, unroll=8)
        def _(c):
            s = c * _L
            key = kv[r, pl.ds(s, _L)]
            xv[r, pl.ds(s, _L)] = jnp.where(
                key > tkey, jnp.float32(1.0), jnp.float32(0.0))

    pltpu.sync_copy(xv, out_hbm.at[pl.ds(base_row, 2)])


def kernel(x):
    batch, emb = x.shape
    k_active = math.ceil(_SPARSITY * emb)
    rows_per_worker = batch // 32
    n_chunks = emb // _L
    mesh = plsc.VectorSubcoreMesh(core_axis_name="c", subcore_axis_name="s")
    body = functools.partial(_sc_body, n_chunks, k_active)
    f = pl.kernel(
        body,
        mesh=mesh,
        compiler_params=pltpu.CompilerParams(needs_layout_passes=False),
        out_type=jax.ShapeDtypeStruct((batch, emb), jnp.float32),
        scratch_types=[
            pltpu.VMEM((rows_per_worker, emb), jnp.float32),
            pltpu.VMEM((rows_per_worker, emb), jnp.int32),
            pltpu.VMEM((256,), jnp.int32),
            pltpu.VMEM((256,), jnp.int32),
            pltpu.VMEM((_L,), jnp.int32),
        ],
    )
    return f(x)


# speedup vs baseline: 1.9791x; 1.0996x over previous
import functools
import math

import jax
import jax.numpy as jnp
from jax import lax
from jax.experimental import pallas as pl
from jax.experimental.pallas import tpu as pltpu
from jax.experimental.pallas import tpu_sc as plsc

_SPARSITY = 0.05
_L = 16
_INT_MIN = -2147483648


def _splat(v, dtype=jnp.int32):
    return jnp.full((_L,), v, dtype=dtype)


def _to_key(i):
    return jnp.where(i < 0, i ^ jnp.int32(0x7FFFFFFF), i)


def _zero256(ref):
    z = jnp.zeros((_L,), jnp.int32)
    for c in range(16):
        ref[pl.ds(c * _L, _L)] = z


def _locate(hist, sufr, tots, rank):
    ids0 = lax.iota(jnp.int32, _L)
    lane0 = ids0 == _splat(0)

    @plsc.parallel_loop(0, 16, unroll=4)
    def _(c):
        h = hist[pl.ds(c * _L, _L)]
        suf = jnp.flip(jnp.cumsum(jnp.flip(h)))
        sufr[pl.ds(c * _L, _L)] = suf
        plsc.store_scatter(tots, [_splat(c)], suf, mask=lane0)

    tv = tots[...]
    ts = jnp.flip(jnp.cumsum(jnp.flip(tv)))
    above_ch = ts - tv
    rank_s = _splat(rank)
    c_best = jnp.max(jnp.where(ts >= rank_s, ids0, _splat(-1)))
    above_c = jnp.max(
        jnp.where(ids0 == _splat(c_best), above_ch, _splat(_INT_MIN)))
    suf_w = sufr[pl.ds(c_best * _L, _L)] + _splat(above_c)
    h = hist[pl.ds(c_best * _L, _L)]
    ids = ids0 + c_best * _L
    b_best = jnp.max(jnp.where(suf_w >= rank_s, ids, _splat(-1)))
    sel = ids == _splat(b_best)
    above = jnp.max(jnp.where(sel, suf_w - h, _splat(-1)))
    h_at = jnp.max(jnp.where(sel, h, _splat(-1)))
    return b_best, above, h_at


def _sc_body(n_chunks, rows_per_worker, k_active, x_hbm, out_hbm, xv, kv, hist,
             sufr, tots):
    wid = lax.axis_index("s") * 2 + lax.axis_index("c")
    base_row = wid * rows_per_worker
    pltpu.sync_copy(x_hbm.at[pl.ds(base_row, rows_per_worker)], xv)

    ones = jnp.ones((_L,), jnp.int32)

    for r in range(rows_per_worker):
        _zero256(hist)

        @plsc.parallel_loop(0, n_chunks, unroll=8)
        def _(c):
            s = c * _L
            xi = lax.bitcast_convert_type(xv[r, pl.ds(s, _L)], jnp.int32)
            key = _to_key(xi)
            kv[r, pl.ds(s, _L)] = key
            plsc.addupdate_scatter(hist, [(key >> 24) + 128], ones)

        b1, above1, _h1 = _locate(hist, sufr, tots, jnp.int32(k_active))
        rank2 = jnp.int32(k_active) - above1

        p1v = b1 - 128

        def hist_level(prefix, pshift, dshift):
            _zero256(hist)
            ps = _splat(prefix)

            @plsc.parallel_loop(0, n_chunks, unroll=8)
            def _(c):
                key = kv[r, pl.ds(c * _L, _L)]
                m = (key >> pshift) == ps
                plsc.addupdate_scatter(
                    hist, [(key >> dshift) & 255], ones, mask=m)

        hist_level(p1v, 24, 16)
        b2, above2, _h2 = _locate(hist, sufr, tots, rank2)
        rank3 = rank2 - above2
        p2v = (p1v << 8) | b2

        hist_level(p2v, 16, 8)
        b3, above3, _h3 = _locate(hist, sufr, tots, rank3)
        rank4 = rank3 - above3
        p3v = (p2v << 8) | b3

        hist_level(p3v, 8, 0)
        b4, above4, h4 = _locate(hist, sufr, tots, rank4)

        v_key = (p3v << 8) | b4
        c_ge = above1 + above2 + above3 + above4 + h4

        vks = _splat(v_key)

        @plsc.parallel_loop(0, n_chunks, unroll=8, carry=_splat(_INT_MIN))
        def mx(c, acc):
            key = kv[r, pl.ds(c * _L, _L)]
            return jnp.maximum(acc, jnp.where(key < vks, key, _splat(_INT_MIN)))

        v2_key = jnp.where(c_ge >= jnp.int32(k_active + 1), v_key, jnp.max(mx))

        f1 = lax.bitcast_convert_type(_to_key(vks), jnp.float32)
        f2 = lax.bitcast_convert_type(_to_key(_splat(v2_key)), jnp.float32)
        thr = (f1 + f2) * jnp.float32(0.5)
        tkey = _to_key(lax.bitcast_convert_type(thr, jnp.int32))

        @plsc.parallel_loop(0, n_chunks, unroll=8)
        def _(c):
            s = c * _L
            key = kv[r, pl.ds(s, _L)]
            xv[r, pl.ds(s, _L)] = jnp.where(
                key > tkey, jnp.float32(1.0), jnp.float32(0.0))

    pltpu.sync_copy(xv, out_hbm.at[pl.ds(base_row, rows_per_worker)])


def _sc_kwta(x):
    batch, emb = x.shape
    k_active = math.ceil(_SPARSITY * emb)
    rows_per_worker = batch // 32
    n_chunks = emb // _L
    mesh = plsc.VectorSubcoreMesh(core_axis_name="c", subcore_axis_name="s")
    body = functools.partial(_sc_body, n_chunks, rows_per_worker, k_active)
    f = pl.kernel(
        body,
        mesh=mesh,
        compiler_params=pltpu.CompilerParams(needs_layout_passes=False),
        out_type=jax.ShapeDtypeStruct((batch, emb), jnp.float32),
        scratch_types=[
            pltpu.VMEM((rows_per_worker, emb), jnp.float32),
            pltpu.VMEM((rows_per_worker, emb), jnp.int32),
            pltpu.VMEM((256,), jnp.int32),
            pltpu.VMEM((256,), jnp.int32),
            pltpu.VMEM((_L,), jnp.int32),
        ],
    )
    return f(x)


def _tc_body(k_active, x_ref, out_ref):
    x = x_ref[...]
    n = x.shape[1]
    i = lax.bitcast_convert_type(x, jnp.int32)
    keys = jnp.where(i < 0, i ^ jnp.int32(0x7FFFFFFF), i)
    int_min = jnp.int32(_INT_MIN)
    v0 = jnp.full((x.shape[0], 1), int_min, dtype=jnp.int32)
    c0 = jnp.full((x.shape[0], 1), n, dtype=jnp.int32)

    def step(it, carry):
        v, cnt_v = carry
        bit = lax.shift_left(jnp.int32(1), jnp.int32(31) - it.astype(jnp.int32))
        trial = v + bit
        cnt = jnp.sum((keys >= trial).astype(jnp.int32), axis=1, keepdims=True)
        take = cnt >= k_active
        return jnp.where(take, trial, v), jnp.where(take, cnt, cnt_v)

    v, cnt_v = lax.fori_loop(0, 32, step, (v0, c0))
    vnext = jnp.max(jnp.where(keys < v, keys, int_min), axis=1, keepdims=True)
    vk1 = jnp.where(cnt_v >= k_active + 1, v, vnext)

    def to_f32(s):
        return lax.bitcast_convert_type(
            jnp.where(s < 0, s ^ jnp.int32(0x7FFFFFFF), s), jnp.float32)

    thr = (to_f32(v) + to_f32(vk1)) * 0.5
    out_ref[...] = (x > thr).astype(jnp.float32)


def _tc_kwta(x):
    batch, emb = x.shape
    k_active = math.ceil(_SPARSITY * emb)
    return pl.pallas_call(
        functools.partial(_tc_body, k_active),
        out_shape=jax.ShapeDtypeStruct((batch, emb), jnp.float32),
    )(x)


_TC_ROWS = 32


def kernel(x):
    y_tc = _tc_kwta(x[:_TC_ROWS])
    y_sc = _sc_kwta(x[_TC_ROWS:])
    return jnp.concatenate([y_tc, y_sc], axis=0)


# speedup vs baseline: 2.0859x; 1.0540x over previous
import functools
import math

import jax
import jax.numpy as jnp
from jax import lax
from jax.experimental import pallas as pl
from jax.experimental.pallas import tpu as pltpu
from jax.experimental.pallas import tpu_sc as plsc

_SPARSITY = 0.05
_L = 16
_INT_MIN = -2147483648


def _splat(v, dtype=jnp.int32):
    return jnp.full((_L,), v, dtype=dtype)


def _to_key(i):
    return jnp.where(i < 0, i ^ jnp.int32(0x7FFFFFFF), i)


def _zero256(ref):
    z = jnp.zeros((_L,), jnp.int32)
    for c in range(16):
        ref[pl.ds(c * _L, _L)] = z


def _locate(hist, sufr, tots, rank):
    ids0 = lax.iota(jnp.int32, _L)
    lane0 = ids0 == _splat(0)

    @plsc.parallel_loop(0, 16, unroll=4)
    def _(c):
        h = hist[pl.ds(c * _L, _L)]
        suf = jnp.flip(jnp.cumsum(jnp.flip(h)))
        sufr[pl.ds(c * _L, _L)] = suf
        plsc.store_scatter(tots, [_splat(c)], suf, mask=lane0)

    tv = tots[...]
    ts = jnp.flip(jnp.cumsum(jnp.flip(tv)))
    above_ch = ts - tv
    rank_s = _splat(rank)
    c_best = jnp.max(jnp.where(ts >= rank_s, ids0, _splat(-1)))
    above_c = jnp.max(
        jnp.where(ids0 == _splat(c_best), above_ch, _splat(_INT_MIN)))
    suf_w = sufr[pl.ds(c_best * _L, _L)] + _splat(above_c)
    h = hist[pl.ds(c_best * _L, _L)]
    ids = ids0 + c_best * _L
    b_best = jnp.max(jnp.where(suf_w >= rank_s, ids, _splat(-1)))
    sel = ids == _splat(b_best)
    above = jnp.max(jnp.where(sel, suf_w - h, _splat(-1)))
    h_at = jnp.max(jnp.where(sel, h, _splat(-1)))
    return b_best, above, h_at


def _sc_body(n_chunks, rows_per_worker, row_offset, k_active, x_hbm, out_hbm,
             xv, kv, hist, sufr, tots):
    wid = lax.axis_index("s") * 2 + lax.axis_index("c")
    base_row = row_offset + wid * rows_per_worker
    pltpu.sync_copy(x_hbm.at[pl.ds(base_row, rows_per_worker)], xv)

    ones = jnp.ones((_L,), jnp.int32)

    for r in range(rows_per_worker):
        _zero256(hist)

        @plsc.parallel_loop(0, n_chunks, unroll=8)
        def _(c):
            s = c * _L
            xi = lax.bitcast_convert_type(xv[r, pl.ds(s, _L)], jnp.int32)
            key = _to_key(xi)
            kv[r, pl.ds(s, _L)] = key
            plsc.addupdate_scatter(hist, [(key >> 24) + 128], ones)

        b1, above1, _h1 = _locate(hist, sufr, tots, jnp.int32(k_active))
        rank2 = jnp.int32(k_active) - above1

        p1v = b1 - 128

        def hist_level(prefix, pshift, dshift):
            _zero256(hist)
            ps = _splat(prefix)

            @plsc.parallel_loop(0, n_chunks, unroll=8)
            def _(c):
                key = kv[r, pl.ds(c * _L, _L)]
                m = (key >> pshift) == ps
                plsc.addupdate_scatter(
                    hist, [(key >> dshift) & 255], ones, mask=m)

        hist_level(p1v, 24, 16)
        b2, above2, _h2 = _locate(hist, sufr, tots, rank2)
        rank3 = rank2 - above2
        p2v = (p1v << 8) | b2

        hist_level(p2v, 16, 8)
        b3, above3, _h3 = _locate(hist, sufr, tots, rank3)
        rank4 = rank3 - above3
        p3v = (p2v << 8) | b3

        hist_level(p3v, 8, 0)
        b4, above4, h4 = _locate(hist, sufr, tots, rank4)

        v_key = (p3v << 8) | b4
        c_ge = above1 + above2 + above3 + above4 + h4

        vks = _splat(v_key)

        @plsc.parallel_loop(0, n_chunks, unroll=8, carry=_splat(_INT_MIN))
        def mx(c, acc):
            key = kv[r, pl.ds(c * _L, _L)]
            return jnp.maximum(acc, jnp.where(key < vks, key, _splat(_INT_MIN)))

        v2_key = jnp.where(c_ge >= jnp.int32(k_active + 1), v_key, jnp.max(mx))

        f1 = lax.bitcast_convert_type(_to_key(vks), jnp.float32)
        f2 = lax.bitcast_convert_type(_to_key(_splat(v2_key)), jnp.float32)
        thr = (f1 + f2) * jnp.float32(0.5)
        tkey = _to_key(lax.bitcast_convert_type(thr, jnp.int32))

        @plsc.parallel_loop(0, n_chunks, unroll=8)
        def _(c):
            s = c * _L
            key = kv[r, pl.ds(s, _L)]
            xv[r, pl.ds(s, _L)] = jnp.where(
                key > tkey, jnp.float32(1.0), jnp.float32(0.0))

    pltpu.sync_copy(xv, out_hbm.at[pl.ds(base_row, rows_per_worker)])


def _sc_kwta(x, row_offset=0):
    batch, emb = x.shape
    k_active = math.ceil(_SPARSITY * emb)
    rows_per_worker = (batch - row_offset) // 32
    n_chunks = emb // _L
    mesh = plsc.VectorSubcoreMesh(core_axis_name="c", subcore_axis_name="s")
    body = functools.partial(
        _sc_body, n_chunks, rows_per_worker, row_offset, k_active)
    f = pl.kernel(
        body,
        mesh=mesh,
        compiler_params=pltpu.CompilerParams(needs_layout_passes=False),
        out_type=jax.ShapeDtypeStruct((batch, emb), jnp.float32),
        scratch_types=[
            pltpu.VMEM((rows_per_worker, emb), jnp.float32),
            pltpu.VMEM((rows_per_worker, emb), jnp.int32),
            pltpu.VMEM((256,), jnp.int32),
            pltpu.VMEM((256,), jnp.int32),
            pltpu.VMEM((_L,), jnp.int32),
        ],
    )
    return f(x)


def _tc_thr_body(k_active, x_ref, thr_ref):
    x = x_ref[...]
    n = x.shape[1]
    i = lax.bitcast_convert_type(x, jnp.int32)
    keys = jnp.where(i < 0, i ^ jnp.int32(0x7FFFFFFF), i)
    int_min = jnp.int32(_INT_MIN)
    v0 = jnp.full((x.shape[0], 1), int_min, dtype=jnp.int32)
    c0 = jnp.full((x.shape[0], 1), n, dtype=jnp.int32)

    def step(it, carry):
        v, cnt_v = carry
        bit = lax.shift_left(jnp.int32(1), jnp.int32(31) - it.astype(jnp.int32))
        trial = v + bit
        cnt = jnp.sum((keys >= trial).astype(jnp.int32), axis=1, keepdims=True)
        take = cnt >= k_active
        return jnp.where(take, trial, v), jnp.where(take, cnt, cnt_v)

    v, cnt_v = lax.fori_loop(0, 32, step, (v0, c0))
    vnext = jnp.max(jnp.where(keys < v, keys, int_min), axis=1, keepdims=True)
    vk1 = jnp.where(cnt_v >= k_active + 1, v, vnext)

    def to_f32(s):
        return lax.bitcast_convert_type(
            jnp.where(s < 0, s ^ jnp.int32(0x7FFFFFFF), s), jnp.float32)

    thr_ref[...] = (to_f32(v) + to_f32(vk1)) * 0.5


def _tc_merge_body(x_ref, thr_ref, _ysc_ref, out_ref):
    out_ref[...] = (x_ref[...] > thr_ref[...]).astype(jnp.float32)


_TC_ROWS = 32


def kernel(x):
    batch, emb = x.shape
    k_active = math.ceil(_SPARSITY * emb)
    y_sc = _sc_kwta(x, _TC_ROWS)
    thr = pl.pallas_call(
        functools.partial(_tc_thr_body, k_active),
        grid=(1,),
        in_specs=[pl.BlockSpec((_TC_ROWS, emb), lambda i: (0, 0))],
        out_specs=pl.BlockSpec((_TC_ROWS, 1), lambda i: (0, 0)),
        out_shape=jax.ShapeDtypeStruct((_TC_ROWS, 1), jnp.float32),
    )(x)
    out = pl.pallas_call(
        _tc_merge_body,
        grid=(1,),
        in_specs=[
            pl.BlockSpec((_TC_ROWS, emb), lambda i: (0, 0)),
            pl.BlockSpec((_TC_ROWS, 1), lambda i: (0, 0)),
            pl.BlockSpec(memory_space=pltpu.MemorySpace.HBM),
        ],
        out_specs=pl.BlockSpec((_TC_ROWS, emb), lambda i: (0, 0)),
        out_shape=jax.ShapeDtypeStruct((batch, emb), jnp.float32),
        input_output_aliases={2: 0},
    )(x, thr, y_sc)
    return out


# speedup vs baseline: 2.0920x; 1.0029x over previous
import functools
import math

import jax
import jax.numpy as jnp
from jax import lax
from jax.experimental import pallas as pl
from jax.experimental.pallas import tpu as pltpu
from jax.experimental.pallas import tpu_sc as plsc

_SPARSITY = 0.05
_L = 16
_INT_MIN = -2147483648


def _splat(v, dtype=jnp.int32):
    return jnp.full((_L,), v, dtype=dtype)


def _to_key(i):
    return jnp.where(i < 0, i ^ jnp.int32(0x7FFFFFFF), i)


def _zero256(ref):
    z = jnp.zeros((_L,), jnp.int32)
    for c in range(16):
        ref[pl.ds(c * _L, _L)] = z


def _locate(hist, sufr, tots, rank):
    ids0 = lax.iota(jnp.int32, _L)
    lane0 = ids0 == _splat(0)

    @plsc.parallel_loop(0, 16, unroll=4)
    def _(c):
        h = hist[pl.ds(c * _L, _L)]
        suf = jnp.flip(jnp.cumsum(jnp.flip(h)))
        sufr[pl.ds(c * _L, _L)] = suf
        plsc.store_scatter(tots, [_splat(c)], suf, mask=lane0)

    tv = tots[...]
    ts = jnp.flip(jnp.cumsum(jnp.flip(tv)))
    above_ch = ts - tv
    rank_s = _splat(rank)
    c_best = jnp.max(jnp.where(ts >= rank_s, ids0, _splat(-1)))
    above_c = jnp.max(
        jnp.where(ids0 == _splat(c_best), above_ch, _splat(_INT_MIN)))
    suf_w = sufr[pl.ds(c_best * _L, _L)] + _splat(above_c)
    h = hist[pl.ds(c_best * _L, _L)]
    ids = ids0 + c_best * _L
    b_best = jnp.max(jnp.where(suf_w >= rank_s, ids, _splat(-1)))
    sel = ids == _splat(b_best)
    above = jnp.max(jnp.where(sel, suf_w - h, _splat(-1)))
    h_at = jnp.max(jnp.where(sel, h, _splat(-1)))
    return b_best, above, h_at


def _sc_body(n_chunks, rows_per_worker, row_offset, k_active, x_hbm, out_hbm,
             xv, kv, hist, sufr, tots):
    wid = lax.axis_index("s") * 2 + lax.axis_index("c")
    base_row = row_offset + wid * rows_per_worker
    pltpu.sync_copy(x_hbm.at[pl.ds(base_row, rows_per_worker)], xv)

    ones = jnp.ones((_L,), jnp.int32)

    for r in range(rows_per_worker):
        _zero256(hist)

        @plsc.parallel_loop(0, n_chunks, unroll=8)
        def _(c):
            s = c * _L
            xi = lax.bitcast_convert_type(xv[r, pl.ds(s, _L)], jnp.int32)
            key = _to_key(xi)
            kv[r, pl.ds(s, _L)] = key
            plsc.addupdate_scatter(hist, [(key >> 24) + 128], ones)

        b1, above1, _h1 = _locate(hist, sufr, tots, jnp.int32(k_active))
        rank2 = jnp.int32(k_active) - above1

        p1v = b1 - 128

        def hist_level(prefix, pshift, dshift):
            _zero256(hist)
            ps = _splat(prefix)

            @plsc.parallel_loop(0, n_chunks, unroll=8)
            def _(c):
                key = kv[r, pl.ds(c * _L, _L)]
                m = (key >> pshift) == ps
                plsc.addupdate_scatter(
                    hist, [(key >> dshift) & 255], ones, mask=m)

        hist_level(p1v, 24, 16)
        b2, above2, _h2 = _locate(hist, sufr, tots, rank2)
        rank3 = rank2 - above2
        p2v = (p1v << 8) | b2

        hist_level(p2v, 16, 8)
        b3, above3, _h3 = _locate(hist, sufr, tots, rank3)
        rank4 = rank3 - above3
        p3v = (p2v << 8) | b3

        hist_level(p3v, 8, 0)
        b4, above4, h4 = _locate(hist, sufr, tots, rank4)

        v_key = (p3v << 8) | b4
        c_ge = above1 + above2 + above3 + above4 + h4

        vks = _splat(v_key)

        @plsc.parallel_loop(0, n_chunks, unroll=8, carry=_splat(_INT_MIN))
        def mx(c, acc):
            key = kv[r, pl.ds(c * _L, _L)]
            return jnp.maximum(acc, jnp.where(key < vks, key, _splat(_INT_MIN)))

        v2_key = jnp.where(c_ge >= jnp.int32(k_active + 1), v_key, jnp.max(mx))

        f1 = lax.bitcast_convert_type(_to_key(vks), jnp.float32)
        f2 = lax.bitcast_convert_type(_to_key(_splat(v2_key)), jnp.float32)
        thr = (f1 + f2) * jnp.float32(0.5)
        tkey = _to_key(lax.bitcast_convert_type(thr, jnp.int32))

        @plsc.parallel_loop(0, n_chunks, unroll=8)
        def _(c):
            s = c * _L
            key = kv[r, pl.ds(s, _L)]
            xv[r, pl.ds(s, _L)] = jnp.where(
                key > tkey, jnp.float32(1.0), jnp.float32(0.0))

    pltpu.sync_copy(xv, out_hbm.at[pl.ds(base_row, rows_per_worker)])


def _sc_kwta(x, row_offset=0):
    batch, emb = x.shape
    k_active = math.ceil(_SPARSITY * emb)
    rows_per_worker = (batch - row_offset) // 32
    n_chunks = emb // _L
    mesh = plsc.VectorSubcoreMesh(core_axis_name="c", subcore_axis_name="s")
    body = functools.partial(
        _sc_body, n_chunks, rows_per_worker, row_offset, k_active)
    f = pl.kernel(
        body,
        mesh=mesh,
        compiler_params=pltpu.CompilerParams(
            needs_layout_passes=False, skip_device_barrier=True),
        out_type=jax.ShapeDtypeStruct((batch, emb), jnp.float32),
        scratch_types=[
            pltpu.VMEM((rows_per_worker, emb), jnp.float32),
            pltpu.VMEM((rows_per_worker, emb), jnp.int32),
            pltpu.VMEM((256,), jnp.int32),
            pltpu.VMEM((256,), jnp.int32),
            pltpu.VMEM((_L,), jnp.int32),
        ],
    )
    return f(x)


def _tc_thr_body(k_active, x_ref, thr_ref):
    x = x_ref[...]
    n = x.shape[1]
    i = lax.bitcast_convert_type(x, jnp.int32)
    keys = jnp.where(i < 0, i ^ jnp.int32(0x7FFFFFFF), i)
    int_min = jnp.int32(_INT_MIN)
    v0 = jnp.full((x.shape[0], 1), int_min, dtype=jnp.int32)
    c0 = jnp.full((x.shape[0], 1), n, dtype=jnp.int32)

    def step(it, carry):
        v, cnt_v = carry
        bit = lax.shift_left(jnp.int32(1), jnp.int32(31) - it.astype(jnp.int32))
        trial = v + bit
        cnt = jnp.sum((keys >= trial).astype(jnp.int32), axis=1, keepdims=True)
        take = cnt >= k_active
        return jnp.where(take, trial, v), jnp.where(take, cnt, cnt_v)

    v, cnt_v = lax.fori_loop(0, 32, step, (v0, c0))
    vnext = jnp.max(jnp.where(keys < v, keys, int_min), axis=1, keepdims=True)
    vk1 = jnp.where(cnt_v >= k_active + 1, v, vnext)

    def to_f32(s):
        return lax.bitcast_convert_type(
            jnp.where(s < 0, s ^ jnp.int32(0x7FFFFFFF), s), jnp.float32)

    thr_ref[...] = (to_f32(v) + to_f32(vk1)) * 0.5


def _tc_merge_body(x_ref, thr_ref, _ysc_ref, out_ref):
    out_ref[...] = (x_ref[...] > thr_ref[...]).astype(jnp.float32)


_TC_ROWS = 32


def kernel(x):
    batch, emb = x.shape
    k_active = math.ceil(_SPARSITY * emb)
    y_sc = _sc_kwta(x, _TC_ROWS)
    thr = pl.pallas_call(
        functools.partial(_tc_thr_body, k_active),
        grid=(1,),
        in_specs=[pl.BlockSpec((_TC_ROWS, emb), lambda i: (0, 0))],
        out_specs=pl.BlockSpec((_TC_ROWS, 1), lambda i: (0, 0)),
        out_shape=jax.ShapeDtypeStruct((_TC_ROWS, 1), jnp.float32),
    )(x)
    out = pl.pallas_call(
        _tc_merge_body,
        grid=(1,),
        in_specs=[
            pl.BlockSpec((_TC_ROWS, emb), lambda i: (0, 0)),
            pl.BlockSpec((_TC_ROWS, 1), lambda i: (0, 0)),
            pl.BlockSpec(memory_space=pltpu.MemorySpace.HBM),
        ],
        out_specs=pl.BlockSpec((_TC_ROWS, emb), lambda i: (0, 0)),
        out_shape=jax.ShapeDtypeStruct((batch, emb), jnp.float32),
        input_output_aliases={2: 0},
    )(x, thr, y_sc)
    return out
